# baseline (speedup 1.0000x reference)
import functools

import jax
import jax.numpy as jnp
from jax import lax
from jax.experimental import pallas as pl
from jax.experimental.pallas import tpu as pltpu
from jax.experimental.pallas import tpu_sc as plsc

N_FIELDS = 26
VOCAB = 100000
DIM = 16
BATCH = 16384

_info = plsc.get_sparse_core_info()
_NC, _NS, _L = _info.num_cores, _info.num_subcores, _info.num_lanes
_NW = _NC * _NS
_BPW = BATCH // _NW
_NJ = _BPW // _L


def _sc_body(xT_hbm, emb_hbm, lin_hbm, out_hbm, xi_v, rows_v, lin_v, acc_v,
             acc2_v, sem_e, sem_l):
    wid = lax.axis_index("s") * _NC + lax.axis_index("c")
    base = wid * _BPW

    pltpu.sync_copy(xT_hbm.at[:, pl.ds(base, _BPW)], xi_v)

    def _off(p, carry):
        f = p // _NJ
        j = p - f * _NJ
        sl = pl.ds(j * _L, _L)
        xi_v[f, sl] = xi_v[f, sl] + f * VOCAB
        return carry

    lax.fori_loop(0, N_FIELDS * _NJ, _off, 0)

    def _zero(j, carry):
        acc_v[pl.ds(j * _L, _L)] = jnp.zeros((_L,), jnp.float32)
        return carry

    lax.fori_loop(0, _NJ, _zero, 0)

    def _field(f, carry):
        pltpu.async_copy(emb_hbm.at[xi_v.at[f]], rows_v, sem_e).wait()
        pltpu.async_copy(lin_hbm.at[xi_v.at[f]], lin_v, sem_l).wait()
        pltpu.sync_copy(rows_v, out_hbm.at[pl.ds(base, _BPW),
                                           pl.ds(f * DIM, DIM)])

        def _acc(j, c):
            sl = pl.ds(j * _L, _L)
            acc_v[sl] = acc_v[sl] + lin_v[sl]
            return c

        lax.fori_loop(0, _NJ, _acc, 0)
        return carry

    lax.fori_loop(0, N_FIELDS, _field, 0)

    def _stage(j, carry):
        rows = lax.iota(jnp.int32, _L) + j * _L
        cols = jnp.zeros((_L,), jnp.int32)
        plsc.store_scatter(acc2_v, [rows, cols], acc_v[pl.ds(j * _L, _L)])
        return carry

    lax.fori_loop(0, _NJ, _stage, 0)
    pltpu.sync_copy(acc2_v, out_hbm.at[pl.ds(base, _BPW),
                                       pl.ds(N_FIELDS * DIM, 1)])


@jax.jit
def _sc_call(xT, emb_flat, lin_flat):
    mesh = plsc.VectorSubcoreMesh(core_axis_name="c", subcore_axis_name="s")
    return pl.kernel(
        _sc_body,
        mesh=mesh,
        compiler_params=pltpu.CompilerParams(use_tc_tiling_on_sc=False,
                                             needs_layout_passes=False),
        out_type=jax.ShapeDtypeStruct((BATCH, N_FIELDS * DIM + 1), jnp.float32),
        scratch_types=[
            pltpu.VMEM((N_FIELDS, _BPW), jnp.int32),
            pltpu.VMEM((_BPW, DIM), jnp.float32),
            pltpu.VMEM((_BPW,), jnp.float32),
            pltpu.VMEM((_BPW,), jnp.float32),
            pltpu.VMEM((_BPW, 1), jnp.float32),
            pltpu.SemaphoreType.DMA,
            pltpu.SemaphoreType.DMA,
        ],
    )(xT, emb_flat, lin_flat)


def kernel(x, emb_tables, lin_tables):
    xT = x.T
    emb_flat = emb_tables.reshape(N_FIELDS * VOCAB, DIM)
    lin_flat = lin_tables.reshape(N_FIELDS * VOCAB)
    return _sc_call(xT, emb_flat, lin_flat)


# speedup vs baseline: 1.0125x; 1.0125x over previous
import functools

import jax
import jax.numpy as jnp
from jax import lax
from jax.experimental import pallas as pl
from jax.experimental.pallas import tpu as pltpu
from jax.experimental.pallas import tpu_sc as plsc

N_FIELDS = 26
VOCAB = 100000
DIM = 16
BATCH = 16384
OUT_D = N_FIELDS * DIM + 1

_info = plsc.get_sparse_core_info()
_NC, _NS, _L = _info.num_cores, _info.num_subcores, _info.num_lanes
_NW = _NC * _NS
_BPW = BATCH // _NW
_NJ = _BPW // _L


def _sc_body(xT_hbm, emb_hbm, lin_hbm, out_hbm, xi_v, rows0_v, rows1_v,
             lin0_v, lin1_v, tb0_v, tb1_v, acc_v, se0, se1, sl0, sl1,
             sw0, sw1):
    wid = lax.axis_index("s") * _NC + lax.axis_index("c")
    base = wid * _BPW

    pltpu.sync_copy(xT_hbm.at[:, pl.ds(base, _BPW)], xi_v)

    def _off(p, carry):
        f = p // _NJ
        j = p - f * _NJ
        sl = pl.ds(j * _L, _L)
        xi_v[f, sl] = xi_v[f, sl] + f * VOCAB
        return carry

    lax.fori_loop(0, N_FIELDS * _NJ, _off, 0)

    def _zero(j, carry):
        acc_v[0, pl.ds(j * _L, _L)] = jnp.zeros((_L,), jnp.float32)
        return carry

    lax.fori_loop(0, _NJ, _zero, 0)

    lane = lax.iota(jnp.int32, _L)

    def _fire(f, rows_v, lin_v, sem_e, sem_l):
        cpe = pltpu.async_copy(emb_hbm.at[xi_v.at[f]], rows_v, sem_e)
        cpl = pltpu.async_copy(lin_hbm.at[xi_v.at[f]], lin_v, sem_l)
        return cpe, cpl

    def _drain(f, rows_v, lin_v, tb_v, sem_e, sem_l, sem_w):
        pltpu.make_async_copy(emb_hbm.at[xi_v.at[f]], rows_v, sem_e).wait()
        pltpu.make_async_copy(lin_hbm.at[xi_v.at[f]], lin_v, sem_l).wait()

        def _chunk(c, carry):
            rr = c * _L + lane
            sl = pl.ds(c * _L, _L)
            for d in range(DIM):
                cc = jnp.full((_L,), d, jnp.int32)
                tb_v[d, sl] = plsc.load_gather(rows_v, [rr, cc])
            acc_v[0, sl] = acc_v[0, sl] + lin_v[sl]
            return carry

        lax.fori_loop(0, _NJ, _chunk, 0)
        return pltpu.async_copy(
            tb_v, out_hbm.at[pl.ds(f * DIM, DIM), pl.ds(base, _BPW)], sem_w)

    _fire(0, rows0_v, lin0_v, se0, sl0)

    def _pair(i, carry):
        f0 = 2 * i
        pl.when(f0 + 1 < N_FIELDS)(
            lambda: _fire(f0 + 1, rows1_v, lin1_v, se1, sl1) and None)
        w0 = _drain(f0, rows0_v, lin0_v, tb0_v, se0, sl0, sw0)
        pl.when(f0 + 2 < N_FIELDS)(
            lambda: _fire(f0 + 2, rows0_v, lin0_v, se0, sl0) and None)
        w1 = _drain(f0 + 1, rows1_v, lin1_v, tb1_v, se1, sl1, sw1)
        w0.wait()
        w1.wait()
        return carry

    lax.fori_loop(0, N_FIELDS // 2, _pair, 0)

    pltpu.sync_copy(acc_v, out_hbm.at[pl.ds(N_FIELDS * DIM, 1),
                                      pl.ds(base, _BPW)])


@jax.jit
def _sc_call(xT, emb_flat, lin_flat):
    mesh = plsc.VectorSubcoreMesh(core_axis_name="c", subcore_axis_name="s")
    return pl.kernel(
        _sc_body,
        mesh=mesh,
        compiler_params=pltpu.CompilerParams(use_tc_tiling_on_sc=False,
                                             needs_layout_passes=False),
        out_type=jax.ShapeDtypeStruct((OUT_D, BATCH), jnp.float32),
        scratch_types=[
            pltpu.VMEM((N_FIELDS, _BPW), jnp.int32),
            pltpu.VMEM((_BPW, DIM), jnp.float32),
            pltpu.VMEM((_BPW, DIM), jnp.float32),
            pltpu.VMEM((_BPW,), jnp.float32),
            pltpu.VMEM((_BPW,), jnp.float32),
            pltpu.VMEM((DIM, _BPW), jnp.float32),
            pltpu.VMEM((DIM, _BPW), jnp.float32),
            pltpu.VMEM((1, _BPW), jnp.float32),
            pltpu.SemaphoreType.DMA,
            pltpu.SemaphoreType.DMA,
            pltpu.SemaphoreType.DMA,
            pltpu.SemaphoreType.DMA,
            pltpu.SemaphoreType.DMA,
            pltpu.SemaphoreType.DMA,
        ],
    )(xT, emb_flat, lin_flat)


def kernel(x, emb_tables, lin_tables):
    xT = x.T
    emb_flat = emb_tables.reshape(N_FIELDS * VOCAB, DIM)
    lin_flat = lin_tables.reshape(N_FIELDS * VOCAB)
    outT = _sc_call(xT, emb_flat, lin_flat)
    return outT.T


# speedup vs baseline: 1.5642x; 1.5448x over previous
import jax
import jax.numpy as jnp
from jax import lax
from jax.experimental import pallas as pl
from jax.experimental.pallas import tpu as pltpu
from jax.experimental.pallas import tpu_sc as plsc

N_FIELDS = 26
VOCAB = 100000
DIM = 16
BATCH = 16384
OUT_D = N_FIELDS * DIM + 1

_info = plsc.get_sparse_core_info()
_NC, _NS, _L = _info.num_cores, _info.num_subcores, _info.num_lanes
_FPC = N_FIELDS // _NC
_BPT = BATCH // _NS
_NJ = _BPT // _L

_W = 384
_NFULL = VOCAB // _W
_WLAST = VOCAB - _NFULL * _W
_NWIN = _NFULL + 1
_NTASK = _FPC * _NWIN
_KMAX = (_NTASK + _NS - 1) // _NS

_SB = 128
_NSB = _BPT // _SB
_LB = BATCH // _NC // _NS
_NLSB = _LB // _SB
_LROWS = (VOCAB + 127) // 128 + 1


def _sc_body(xT_hbm, emb_hbm, lin_hbm, out_hbm, scr_hbm, lscr_hbm, se0, se1,
             sw0, sw1):
    c = lax.axis_index("c")
    s = lax.axis_index("s")
    lane = lax.iota(jnp.int32, _L)
    lane_hi = lax.shift_right_logical(lane, 3)
    lane_lo16 = (lane & jnp.full((_L,), 7, jnp.int32)) * DIM

    def _phase_a(in_v, ob_v):
        def _task(k, carry):
            task = s + _NS * k

            @pl.when(task < _NTASK)
            def _():
                f_l = task // _NWIN
                w = task - f_l * _NWIN
                f = c * _FPC + f_l
                wlo = w * _W

                def _do(rd_ext, wr_ext):
                    pltpu.sync_copy(
                        emb_hbm.at[pl.ds(f, 1), :, pl.ds(wlo, rd_ext)],
                        in_v.at[:, :, pl.ds(0, rd_ext)])

                    def _tr(j, cc):
                        v0 = j * _L
                        sl = pl.ds(v0, _L)
                        rr = v0 // 8 + lane_hi
                        for d in range(DIM):
                            plsc.store_scatter(ob_v, [rr, lane_lo16 + d],
                                               in_v[0, d, sl])
                        return cc

                    lax.fori_loop(0, wr_ext // _L, _tr, 0)
                    pltpu.sync_copy(
                        ob_v.at[pl.ds(0, wr_ext // 8), :],
                        scr_hbm.at[pl.ds((f * VOCAB + wlo) // 8,
                                         wr_ext // 8), :])

                @pl.when(w < _NFULL)
                def _():
                    _do(_W, _W)

                @pl.when(w == _NFULL)
                def _():
                    _do(256, _WLAST)

            return carry

        lax.fori_loop(0, _KMAX, _task, 0)

    pl.run_scoped(
        _phase_a,
        pltpu.VMEM((1, DIM, _W), jnp.float32),
        pltpu.VMEM((_W // 8, 8 * DIM), jnp.float32),
    )

    def _phase_a2(li_v, ls_v):
        def _task(k, carry):
            task = s + _NS * k
            ntask = 4 * _NWIN

            @pl.when(task < ntask)
            def _():
                g = task // _NWIN
                w = task - g * _NWIN
                wlo = w * _W

                def _do(rd_ext, wr_ext, nw):
                    pltpu.sync_copy(
                        lin_hbm.at[pl.ds(8 * g, 8), pl.ds(wlo, rd_ext)],
                        li_v.at[:, pl.ds(0, rd_ext)])
                    nrow = jnp.where(g == 3, 2, 8)

                    def _row(r, cc):
                        f = 8 * g + r

                        @pl.when(r < nrow)
                        def _():
                            def _cp(kk, c2):
                                ls_v[kk // 8, pl.ds((kk % 8) * _L, _L)] = \
                                    li_v[r, pl.ds(kk * _L, _L)]
                                return c2

                            lax.fori_loop(0, wr_ext // _L, _cp, 0)
                            pltpu.sync_copy(
                                ls_v.at[pl.ds(0, nw), :],
                                lscr_hbm.at[pl.ds(f * _LROWS + wlo // 128,
                                                  nw), :])

                        return cc

                    lax.fori_loop(0, 8, _row, 0)

                @pl.when(w < _NFULL)
                def _():
                    _do(_W, _W, _W // 128)

                @pl.when(w == _NFULL)
                def _():
                    _do(256, 256, 2)

            return carry

        lax.fori_loop(0, (4 * _NWIN + _NS - 1) // _NS, _task, 0)

    pl.run_scoped(
        _phase_a2,
        pltpu.VMEM((8, _W), jnp.float32),
        pltpu.VMEM((_W // 128, 128), jnp.float32),
    )
    plsc.subcore_barrier()

    def _phase_b(xi_v, xr0_v, xr1_v, lr0_v, lr1_v, tb0_v, tb1_v, acc_v):
        base = s * _BPT
        pltpu.sync_copy(xT_hbm.at[:, pl.ds(base, _BPT)], xi_v)
        loff = c * _LB
        lbase = base + loff

        def _field_body(f_l, tb_v, sem_w):
            f = c * _FPC + f_l

            @pl.when(f_l >= 2)
            def _():
                pltpu.make_async_copy(
                    tb_v, out_hbm.at[pl.ds((f - 2) * DIM, DIM),
                                     pl.ds(base, _BPT)], sem_w).wait()

            def _gfire(sb, lr_v, xr_v, sem):
                def _ridx(j, cc):
                    sl = pl.ds(j * _L, _L)
                    xr_v[sl] = lax.shift_right_logical(
                        xi_v[f, pl.ds(sb * _SB + j * _L, _L)] + f * VOCAB,
                        3)
                    return cc

                lax.fori_loop(0, _SB // _L, _ridx, 0)
                pltpu.async_copy(scr_hbm.at[xr_v], lr_v, sem)

            def _gdrain(sb, lr_v, xr_v, sem):
                pltpu.make_async_copy(scr_hbm.at[xr_v], lr_v, sem).wait()
                for j in range(_SB // _L):
                    sl16 = pl.ds(sb * _SB + j * _L, _L)
                    vm = (xi_v[f, sl16] & jnp.full((_L,), 7,
                                                   jnp.int32)) * DIM
                    rr = j * _L + lane
                    for d in range(DIM):
                        tb_v[d, sl16] = plsc.load_gather(lr_v, [rr, vm + d])

            _gfire(0, lr0_v, xr0_v, se0)
            _gfire(1, lr1_v, xr1_v, se1)
            for sb in range(_NSB):
                _gdrain(sb, lr0_v if sb % 2 == 0 else lr1_v,
                        xr0_v if sb % 2 == 0 else xr1_v,
                        se0 if sb % 2 == 0 else se1)
                if sb + 2 < _NSB:
                    _gfire(sb + 2, lr0_v if sb % 2 == 0 else lr1_v,
                           xr0_v if sb % 2 == 0 else xr1_v,
                           se0 if sb % 2 == 0 else se1)

            pltpu.async_copy(tb_v, out_hbm.at[pl.ds(f * DIM, DIM),
                                              pl.ds(base, _BPT)], sem_w)

        def _pair(i, carry):
            _field_body(2 * i, tb0_v, sw0)
            _field_body(2 * i + 1, tb1_v, sw1)
            return carry

        lax.fori_loop(0, _FPC // 2, _pair, 0)
        _field_body(_FPC - 1, tb0_v, sw0)
        f_last = c * _FPC + _FPC - 1
        pltpu.make_async_copy(
            tb0_v, out_hbm.at[pl.ds(f_last * DIM, DIM),
                              pl.ds(base, _BPT)], sw0).wait()
        pltpu.make_async_copy(
            tb1_v, out_hbm.at[pl.ds((f_last - 1) * DIM, DIM),
                              pl.ds(base, _BPT)], sw1).wait()

        def _zero(j, carry):
            acc_v[0, pl.ds(j * _L, _L)] = jnp.zeros((_L,), jnp.float32)
            return carry

        lax.fori_loop(0, _LB // _L, _zero, 0)

        def _lfield(fi, carry):
            for sb in range(_NLSB):
                def _ridx(j, cc):
                    sl = pl.ds(j * _L, _L)
                    xr0_v[sl] = fi * _LROWS + lax.shift_right_logical(
                        xi_v[fi, pl.ds(loff + sb * _SB + j * _L, _L)], 7)
                    return cc

                lax.fori_loop(0, _SB // _L, _ridx, 0)
                pltpu.async_copy(lscr_hbm.at[xr0_v], lr0_v, se0).wait()
                for j in range(_SB // _L):
                    sl16 = pl.ds(sb * _SB + j * _L, _L)
                    col = xi_v[fi, pl.ds(loff + sb * _SB + j * _L, _L)] \
                        & jnp.full((_L,), 127, jnp.int32)
                    rr = j * _L + lane
                    acc_v[0, sl16] = acc_v[0, sl16] + plsc.load_gather(
                        lr0_v, [rr, col])
            return carry

        lax.fori_loop(0, N_FIELDS, _lfield, 0)
        pltpu.sync_copy(acc_v, out_hbm.at[pl.ds(N_FIELDS * DIM, 1),
                                          pl.ds(lbase, _LB)])

    pl.run_scoped(
        _phase_b,
        pltpu.VMEM((N_FIELDS, _BPT), jnp.int32),
        pltpu.VMEM((_SB,), jnp.int32),
        pltpu.VMEM((_SB,), jnp.int32),
        pltpu.VMEM((_SB, 8 * DIM), jnp.float32),
        pltpu.VMEM((_SB, 8 * DIM), jnp.float32),
        pltpu.VMEM((DIM, _BPT), jnp.float32),
        pltpu.VMEM((DIM, _BPT), jnp.float32),
        pltpu.VMEM((1, _LB), jnp.float32),
    )


@jax.jit
def _sc_call(xT, emb3, lin2):
    mesh = plsc.VectorSubcoreMesh(core_axis_name="c", subcore_axis_name="s")
    return pl.kernel(
        _sc_body,
        mesh=mesh,
        compiler_params=pltpu.CompilerParams(needs_layout_passes=False),
        out_type=jax.ShapeDtypeStruct((OUT_D, BATCH), jnp.float32),
        scratch_types=[
            pltpu.MemorySpace.HBM((N_FIELDS * VOCAB // 8, 8 * DIM),
                                  jnp.float32),
            pltpu.MemorySpace.HBM((N_FIELDS * _LROWS, 128), jnp.float32),
            pltpu.SemaphoreType.DMA,
            pltpu.SemaphoreType.DMA,
            pltpu.SemaphoreType.DMA,
            pltpu.SemaphoreType.DMA,
        ],
    )(xT, emb3, lin2)


def kernel(x, emb_tables, lin_tables):
    xT = x.T
    emb3 = jnp.transpose(emb_tables, (0, 2, 1))
    lin2 = lin_tables.reshape(N_FIELDS, VOCAB)
    outT = _sc_call(xT, emb3, lin2)
    return outT.T


# speedup vs baseline: 1.7993x; 1.1503x over previous
import jax
import jax.numpy as jnp
from jax import lax
from jax.experimental import pallas as pl
from jax.experimental.pallas import tpu as pltpu
from jax.experimental.pallas import tpu_sc as plsc

N_FIELDS = 26
VOCAB = 100000
DIM = 16
BATCH = 16384
OUT_D = N_FIELDS * DIM + 1

_info = plsc.get_sparse_core_info()
_NC, _NS, _L = _info.num_cores, _info.num_subcores, _info.num_lanes
_FPC = N_FIELDS // _NC
_BPT = BATCH // _NS
_NJ = _BPT // _L

_W = 256
_NFULL = VOCAB // _W
_WLAST = VOCAB - _NFULL * _W
_NWIN = _NFULL + 1
_NTASK = _FPC * _NWIN
_KMAX = (_NTASK + _NS - 1) // _NS

_SB = 128
_NSB = _BPT // _SB
_LB = BATCH // _NC // _NS
_NLSB = _LB // _SB
_LROWS = (VOCAB + 127) // 128 + 1


def _sc_body(xT_hbm, emb_hbm, lin_hbm, out_hbm, scr_hbm, lscr_hbm, se0, se1,
             sw0, sw1):
    c = lax.axis_index("c")
    s = lax.axis_index("s")
    lane = lax.iota(jnp.int32, _L)
    lane_hi = lax.shift_right_logical(lane, 3)
    lane_lo16 = (lane & jnp.full((_L,), 7, jnp.int32)) * DIM

    def _phase_a(in0_v, in1_v, ob0_v, ob1_v, sa0, sa1, sb0, sb1):
        def _src(k):
            task = s + _NS * k
            f_l = task // _NWIN
            w = task - f_l * _NWIN
            f = c * _FPC + f_l
            return task, f, w, w * _W

        def _fire_in(k, in_v, sem):
            task, f, w, wlo = _src(k)

            @pl.when(task < _NTASK)
            def _():
                pltpu.async_copy(
                    emb_hbm.at[pl.ds(f, 1), :, pl.ds(wlo, _W)], in_v, sem)

        def _proc(k, in_v, ob_v, sem_in, sem_out):
            task, f, w, wlo = _src(k)

            @pl.when(task < _NTASK)
            def _():
                pltpu.make_async_copy(
                    emb_hbm.at[pl.ds(f, 1), :, pl.ds(wlo, _W)], in_v,
                    sem_in).wait()

            def _do(wr_ext):
                def _tr(j, cc):
                    v0 = j * _L
                    sl = pl.ds(v0, _L)
                    rr = v0 // 8 + lane_hi
                    for d in range(DIM):
                        plsc.store_scatter(ob_v, [rr, lane_lo16 + d],
                                           in_v[0, d, sl])
                    return cc

                lax.fori_loop(0, wr_ext // _L, _tr, 0)
                pltpu.async_copy(
                    ob_v.at[pl.ds(0, wr_ext // 8), :],
                    scr_hbm.at[pl.ds((f * VOCAB + wlo) // 8,
                                     wr_ext // 8), :], sem_out)

            def _wait_out(wr_ext):
                task2, f2, w2, wlo2 = _src(k - 2)
                pltpu.make_async_copy(
                    ob_v.at[pl.ds(0, wr_ext // 8), :],
                    scr_hbm.at[pl.ds((f2 * VOCAB + wlo2) // 8,
                                     wr_ext // 8), :], sem_out).wait()

            @pl.when(k >= 2)
            def _():
                _, _, w2, _ = _src(k - 2)
                pl.when(w2 < _NFULL)(lambda: _wait_out(_W))
                pl.when(w2 == _NFULL)(lambda: _wait_out(_WLAST))

            @pl.when(task < _NTASK)
            def _():
                pl.when(w < _NFULL)(lambda: _do(_W))
                pl.when(w == _NFULL)(lambda: _do(_WLAST))

        _fire_in(0, in0_v, sa0)
        _fire_in(1, in1_v, sa1)

        def _pairs(i, carry):
            k0 = 2 * i
            _proc(k0, in0_v, ob0_v, sa0, sb0)
            _fire_in(k0 + 2, in0_v, sa0)
            _proc(k0 + 1, in1_v, ob1_v, sa1, sb1)
            _fire_in(k0 + 3, in1_v, sa1)
            return carry

        lax.fori_loop(0, (_KMAX + 1) // 2, _pairs, 0)
        def _wait_last(k, ob_v, sem_out):
            task, f, w, wlo = _src(k)

            @pl.when(task < _NTASK)
            def _():
                def _w(wr_ext):
                    pltpu.make_async_copy(
                        ob_v.at[pl.ds(0, wr_ext // 8), :],
                        scr_hbm.at[pl.ds((f * VOCAB + wlo) // 8,
                                         wr_ext // 8), :], sem_out).wait()

                pl.when(w < _NFULL)(lambda: _w(_W))
                pl.when(w == _NFULL)(lambda: _w(_WLAST))

        klast = 2 * ((_KMAX + 1) // 2) - 1
        _wait_last(klast - 1, ob0_v, sb0)
        _wait_last(klast, ob1_v, sb1)

    pl.run_scoped(
        _phase_a,
        pltpu.VMEM((1, DIM, _W), jnp.float32),
        pltpu.VMEM((1, DIM, _W), jnp.float32),
        pltpu.VMEM((_W // 8, 8 * DIM), jnp.float32),
        pltpu.VMEM((_W // 8, 8 * DIM), jnp.float32),
        pltpu.SemaphoreType.DMA,
        pltpu.SemaphoreType.DMA,
        pltpu.SemaphoreType.DMA,
        pltpu.SemaphoreType.DMA,
    )

    def _phase_a2(li_v, ls_v):
        def _task(k, carry):
            task = s + _NS * k
            ntask = 4 * _NWIN

            @pl.when(task < ntask)
            def _():
                g = task // _NWIN
                w = task - g * _NWIN
                wlo = w * _W

                def _do(rd_ext, wr_ext, nw):
                    pltpu.sync_copy(
                        lin_hbm.at[pl.ds(8 * g, 8), pl.ds(wlo, rd_ext)],
                        li_v.at[:, pl.ds(0, rd_ext)])
                    nrow = jnp.where(g == 3, 2, 8)

                    def _row(r, cc):
                        f = 8 * g + r

                        @pl.when(r < nrow)
                        def _():
                            def _cp(kk, c2):
                                ls_v[kk // 8, pl.ds((kk % 8) * _L, _L)] = \
                                    li_v[r, pl.ds(kk * _L, _L)]
                                return c2

                            lax.fori_loop(0, wr_ext // _L, _cp, 0)
                            pltpu.sync_copy(
                                ls_v.at[pl.ds(0, nw), :],
                                lscr_hbm.at[pl.ds(f * _LROWS + wlo // 128,
                                                  nw), :])

                        return cc

                    lax.fori_loop(0, 8, _row, 0)

                _do(_W, _W, 2)

            return carry

        lax.fori_loop(0, (4 * _NWIN + _NS - 1) // _NS, _task, 0)

    pl.run_scoped(
        _phase_a2,
        pltpu.VMEM((8, _W), jnp.float32),
        pltpu.VMEM((_W // 128, 128), jnp.float32),
    )
    plsc.subcore_barrier()

    def _phase_b(xi_v, xr0_v, xr1_v, lr0_v, lr1_v, tb0_v, acc_v):
        base = s * _BPT
        pltpu.sync_copy(xT_hbm.at[:, pl.ds(base, _BPT)], xi_v)
        loff = c * _LB
        lbase = base + loff

        def _field_body(f_l, carry):
            f = c * _FPC + f_l
            tb_v, sem_w = tb0_v, sw0

            @pl.when(f_l >= 1)
            def _():
                pltpu.make_async_copy(
                    tb_v, out_hbm.at[pl.ds((f - 1) * DIM, DIM),
                                     pl.ds(base, _BPT)], sem_w).wait()

            def _gfire(sb, lr_v, xr_v, sem):
                def _ridx(j, cc):
                    sl = pl.ds(j * _L, _L)
                    xr_v[sl] = lax.shift_right_logical(
                        xi_v[f, pl.ds(sb * _SB + j * _L, _L)] + f * VOCAB,
                        3)
                    return cc

                lax.fori_loop(0, _SB // _L, _ridx, 0)
                pltpu.async_copy(scr_hbm.at[xr_v], lr_v, sem)

            def _gdrain(sb, lr_v, xr_v, sem):
                pltpu.make_async_copy(scr_hbm.at[xr_v], lr_v, sem).wait()
                for j in range(_SB // _L):
                    sl16 = pl.ds(sb * _SB + j * _L, _L)
                    vm = (xi_v[f, sl16] & jnp.full((_L,), 7,
                                                   jnp.int32)) * DIM
                    rr = j * _L + lane
                    for d in range(DIM):
                        tb_v[d, sl16] = plsc.load_gather(lr_v, [rr, vm + d])

            _gfire(0, lr0_v, xr0_v, se0)
            _gfire(1, lr1_v, xr1_v, se1)
            for sb in range(_NSB):
                _gdrain(sb, lr0_v if sb % 2 == 0 else lr1_v,
                        xr0_v if sb % 2 == 0 else xr1_v,
                        se0 if sb % 2 == 0 else se1)
                if sb + 2 < _NSB:
                    _gfire(sb + 2, lr0_v if sb % 2 == 0 else lr1_v,
                           xr0_v if sb % 2 == 0 else xr1_v,
                           se0 if sb % 2 == 0 else se1)

            pltpu.async_copy(tb_v, out_hbm.at[pl.ds(f * DIM, DIM),
                                              pl.ds(base, _BPT)], sem_w)
            return carry

        lax.fori_loop(0, _FPC, _field_body, 0)
        f_last = c * _FPC + _FPC - 1
        pltpu.make_async_copy(
            tb0_v, out_hbm.at[pl.ds(f_last * DIM, DIM),
                              pl.ds(base, _BPT)], sw0).wait()

        def _zero(j, carry):
            acc_v[0, pl.ds(j * _L, _L)] = jnp.zeros((_L,), jnp.float32)
            return carry

        lax.fori_loop(0, _LB // _L, _zero, 0)

        def _lfield(fi, carry):
            for sb in range(_NLSB):
                def _ridx(j, cc):
                    sl = pl.ds(j * _L, _L)
                    xr0_v[sl] = fi * _LROWS + lax.shift_right_logical(
                        xi_v[fi, pl.ds(loff + sb * _SB + j * _L, _L)], 7)
                    return cc

                lax.fori_loop(0, _SB // _L, _ridx, 0)
                pltpu.async_copy(lscr_hbm.at[xr0_v], lr0_v, se0).wait()
                for j in range(_SB // _L):
                    sl16 = pl.ds(sb * _SB + j * _L, _L)
                    col = xi_v[fi, pl.ds(loff + sb * _SB + j * _L, _L)] \
                        & jnp.full((_L,), 127, jnp.int32)
                    rr = j * _L + lane
                    acc_v[0, sl16] = acc_v[0, sl16] + plsc.load_gather(
                        lr0_v, [rr, col])
            return carry

        lax.fori_loop(0, N_FIELDS, _lfield, 0)
        pltpu.sync_copy(acc_v, out_hbm.at[pl.ds(N_FIELDS * DIM, 1),
                                          pl.ds(lbase, _LB)])

    pl.run_scoped(
        _phase_b,
        pltpu.VMEM((N_FIELDS, _BPT), jnp.int32),
        pltpu.VMEM((_SB,), jnp.int32),
        pltpu.VMEM((_SB,), jnp.int32),
        pltpu.VMEM((_SB, 8 * DIM), jnp.float32),
        pltpu.VMEM((_SB, 8 * DIM), jnp.float32),
        pltpu.VMEM((DIM, _BPT), jnp.float32),
        pltpu.VMEM((1, _LB), jnp.float32),
    )


@jax.jit
def _sc_call(xT, emb3, lin2):
    mesh = plsc.VectorSubcoreMesh(core_axis_name="c", subcore_axis_name="s")
    return pl.kernel(
        _sc_body,
        mesh=mesh,
        compiler_params=pltpu.CompilerParams(needs_layout_passes=False),
        out_type=jax.ShapeDtypeStruct((OUT_D, BATCH), jnp.float32),
        scratch_types=[
            pltpu.MemorySpace.HBM((N_FIELDS * VOCAB // 8, 8 * DIM),
                                  jnp.float32),
            pltpu.MemorySpace.HBM((N_FIELDS * _LROWS, 128), jnp.float32),
            pltpu.SemaphoreType.DMA,
            pltpu.SemaphoreType.DMA,
            pltpu.SemaphoreType.DMA,
            pltpu.SemaphoreType.DMA,
        ],
    )(xT, emb3, lin2)


def kernel(x, emb_tables, lin_tables):
    xT = x.T
    emb3 = jnp.transpose(emb_tables, (0, 2, 1))
    lin2 = lin_tables.reshape(N_FIELDS, VOCAB)
    outT = _sc_call(xT, emb3, lin2)
    return outT.T


# speedup vs baseline: 2.2326x; 1.2408x over previous
import jax
import jax.numpy as jnp
from jax import lax
from jax.experimental import pallas as pl
from jax.experimental.pallas import tpu as pltpu
from jax.experimental.pallas import tpu_sc as plsc

N_FIELDS = 26
VOCAB = 100000
DIM = 16
BATCH = 16384
OUT_D = N_FIELDS * DIM + 1

_info = plsc.get_sparse_core_info()
_NC, _NS, _L = _info.num_cores, _info.num_subcores, _info.num_lanes
_FPC = N_FIELDS // _NC
_BPT = BATCH // _NS
_NJ = _BPT // _L

_W = 256
_NFULL = VOCAB // _W
_WLAST = VOCAB - _NFULL * _W
_NWIN = _NFULL + 1
_NTASK = _FPC * _NWIN
_KMAX = (_NTASK + _NS - 1) // _NS

_SB = 128
_NSB = _BPT // _SB
_LB = BATCH // _NC // _NS
_NLSB = _LB // _SB
_LROWS = (VOCAB + 127) // 128 + 1
_W2 = 2048
_NW2 = (VOCAB + _W2 - 1) // _W2


def _sc_body(xT_hbm, emb_hbm, lin_hbm, out_hbm, scr_hbm, lscr_hbm, se0, se1,
             sw0, sw1):
    c = lax.axis_index("c")
    s = lax.axis_index("s")
    lane = lax.iota(jnp.int32, _L)
    lane_hi = lax.shift_right_logical(lane, 3)
    lane_lo16 = (lane & jnp.full((_L,), 7, jnp.int32)) * DIM

    def _phase_a(in0_v, in1_v, ob0_v, ob1_v, sa0, sa1, sb0, sb1):
        def _src(k):
            task = s + _NS * k
            f_l = task // _NWIN
            w = task - f_l * _NWIN
            f = c * _FPC + f_l
            return task, f, w, w * _W

        def _fire_in(k, in_v, sem):
            task, f, w, wlo = _src(k)

            @pl.when(task < _NTASK)
            def _():
                pltpu.async_copy(
                    emb_hbm.at[pl.ds(f, 1), :, pl.ds(wlo, _W)], in_v, sem)

        def _proc(k, in_v, ob_v, sem_in, sem_out):
            task, f, w, wlo = _src(k)

            @pl.when(task < _NTASK)
            def _():
                pltpu.make_async_copy(
                    emb_hbm.at[pl.ds(f, 1), :, pl.ds(wlo, _W)], in_v,
                    sem_in).wait()

            def _do(wr_ext):
                def _tr(j, cc):
                    v0 = j * _L
                    sl = pl.ds(v0, _L)
                    rr = v0 // 8 + lane_hi
                    for d in range(DIM):
                        plsc.store_scatter(ob_v, [rr, lane_lo16 + d],
                                           in_v[0, d, sl])
                    return cc

                lax.fori_loop(0, wr_ext // _L, _tr, 0)
                pltpu.async_copy(
                    ob_v.at[pl.ds(0, wr_ext // 8), :],
                    scr_hbm.at[pl.ds((f * VOCAB + wlo) // 8,
                                     wr_ext // 8), :], sem_out)

            def _wait_out(wr_ext):
                task2, f2, w2, wlo2 = _src(k - 2)
                pltpu.make_async_copy(
                    ob_v.at[pl.ds(0, wr_ext // 8), :],
                    scr_hbm.at[pl.ds((f2 * VOCAB + wlo2) // 8,
                                     wr_ext // 8), :], sem_out).wait()

            @pl.when(k >= 2)
            def _():
                _, _, w2, _ = _src(k - 2)
                pl.when(w2 < _NFULL)(lambda: _wait_out(_W))
                pl.when(w2 == _NFULL)(lambda: _wait_out(_WLAST))

            @pl.when(task < _NTASK)
            def _():
                pl.when(w < _NFULL)(lambda: _do(_W))
                pl.when(w == _NFULL)(lambda: _do(_WLAST))

        _fire_in(0, in0_v, sa0)
        _fire_in(1, in1_v, sa1)

        def _pairs(i, carry):
            k0 = 2 * i
            _proc(k0, in0_v, ob0_v, sa0, sb0)
            _fire_in(k0 + 2, in0_v, sa0)
            _proc(k0 + 1, in1_v, ob1_v, sa1, sb1)
            _fire_in(k0 + 3, in1_v, sa1)
            return carry

        lax.fori_loop(0, (_KMAX + 1) // 2, _pairs, 0)
        def _wait_last(k, ob_v, sem_out):
            task, f, w, wlo = _src(k)

            @pl.when(task < _NTASK)
            def _():
                def _w(wr_ext):
                    pltpu.make_async_copy(
                        ob_v.at[pl.ds(0, wr_ext // 8), :],
                        scr_hbm.at[pl.ds((f * VOCAB + wlo) // 8,
                                         wr_ext // 8), :], sem_out).wait()

                pl.when(w < _NFULL)(lambda: _w(_W))
                pl.when(w == _NFULL)(lambda: _w(_WLAST))

        klast = 2 * ((_KMAX + 1) // 2) - 1
        _wait_last(klast - 1, ob0_v, sb0)
        _wait_last(klast, ob1_v, sb1)

    pl.run_scoped(
        _phase_a,
        pltpu.VMEM((1, DIM, _W), jnp.float32),
        pltpu.VMEM((1, DIM, _W), jnp.float32),
        pltpu.VMEM((_W // 8, 8 * DIM), jnp.float32),
        pltpu.VMEM((_W // 8, 8 * DIM), jnp.float32),
        pltpu.SemaphoreType.DMA,
        pltpu.SemaphoreType.DMA,
        pltpu.SemaphoreType.DMA,
        pltpu.SemaphoreType.DMA,
    )

    def _phase_a2(li_v, ls_v):
        def _task(k, carry):
            task = s + _NS * k
            ntask = 4 * _NW2

            @pl.when(task < ntask)
            def _():
                g = task // _NW2
                w = task - g * _NW2
                wlo = w * _W2

                def _do(rd_ext, nw):
                    pltpu.sync_copy(
                        lin_hbm.at[pl.ds(8 * g, 8), pl.ds(wlo, rd_ext)],
                        li_v.at[:, pl.ds(0, rd_ext)])
                    nrow = jnp.where(g == 3, 2, 8)

                    def _row(r, cc):
                        f = 8 * g + r

                        @pl.when(r < nrow)
                        def _():
                            def _cp(kk, c2):
                                ls_v[kk // 8, pl.ds((kk % 8) * _L, _L)] = \
                                    li_v[r, pl.ds(kk * _L, _L)]
                                return c2

                            lax.fori_loop(0, nw * 8, _cp, 0)
                            pltpu.sync_copy(
                                ls_v.at[pl.ds(0, nw), :],
                                lscr_hbm.at[pl.ds(f * _LROWS + wlo // 128,
                                                  nw), :])

                        return cc

                    lax.fori_loop(0, 8, _row, 0)

                pl.when(w < _NW2 - 1)(lambda: _do(_W2, _W2 // 128))
                pl.when(w == _NW2 - 1)(lambda: _do(1792, 14))

            return carry

        lax.fori_loop(0, (4 * _NW2 + _NS - 1) // _NS, _task, 0)

    pl.run_scoped(
        _phase_a2,
        pltpu.VMEM((8, _W2), jnp.float32),
        pltpu.VMEM((_W2 // 128, 128), jnp.float32),
    )
    plsc.subcore_barrier()

    def _phase_b(xi_v, xr0_v, xr1_v, lr0_v, lr1_v, tb0_v, acc_v):
        base = s * _BPT
        pltpu.sync_copy(xT_hbm.at[:, pl.ds(base, _BPT)], xi_v)
        loff = c * _LB
        lbase = base + loff

        def _field_body(f_l, carry):
            f = c * _FPC + f_l
            tb_v, sem_w = tb0_v, sw0

            @pl.when(f_l >= 1)
            def _():
                pltpu.make_async_copy(
                    tb_v, out_hbm.at[pl.ds((f - 1) * DIM, DIM),
                                     pl.ds(base, _BPT)], sem_w).wait()

            def _gfire(sb, lr_v, xr_v, sem):
                def _ridx(j, cc):
                    sl = pl.ds(j * _L, _L)
                    xr_v[sl] = lax.shift_right_logical(
                        xi_v[f, pl.ds(sb * _SB + j * _L, _L)] + f * VOCAB,
                        3)
                    return cc

                lax.fori_loop(0, _SB // _L, _ridx, 0)
                pltpu.async_copy(scr_hbm.at[xr_v], lr_v, sem)

            def _gdrain(sb, lr_v, xr_v, sem):
                pltpu.make_async_copy(scr_hbm.at[xr_v], lr_v, sem).wait()
                for j in range(_SB // _L):
                    sl16 = pl.ds(sb * _SB + j * _L, _L)
                    vm = (xi_v[f, sl16] & jnp.full((_L,), 7,
                                                   jnp.int32)) * DIM
                    rr = j * _L + lane
                    for d in range(DIM):
                        tb_v[d, sl16] = plsc.load_gather(lr_v, [rr, vm + d])

            _gfire(0, lr0_v, xr0_v, se0)
            _gfire(1, lr1_v, xr1_v, se1)
            for sb in range(_NSB):
                _gdrain(sb, lr0_v if sb % 2 == 0 else lr1_v,
                        xr0_v if sb % 2 == 0 else xr1_v,
                        se0 if sb % 2 == 0 else se1)
                if sb + 2 < _NSB:
                    _gfire(sb + 2, lr0_v if sb % 2 == 0 else lr1_v,
                           xr0_v if sb % 2 == 0 else xr1_v,
                           se0 if sb % 2 == 0 else se1)

            pltpu.async_copy(tb_v, out_hbm.at[pl.ds(f * DIM, DIM),
                                              pl.ds(base, _BPT)], sem_w)
            return carry

        lax.fori_loop(0, _FPC, _field_body, 0)
        f_last = c * _FPC + _FPC - 1
        pltpu.make_async_copy(
            tb0_v, out_hbm.at[pl.ds(f_last * DIM, DIM),
                              pl.ds(base, _BPT)], sw0).wait()

        def _zero(j, carry):
            acc_v[0, pl.ds(j * _L, _L)] = jnp.zeros((_L,), jnp.float32)
            return carry

        lax.fori_loop(0, _LB // _L, _zero, 0)

        _NLT = N_FIELDS * _NLSB

        def _lfire(t, xr_v, sem):
            @pl.when(t < _NLT)
            def _():
                fi = t // _NLSB
                sb = t - fi * _NLSB

                def _ridx(j, cc):
                    sl = pl.ds(j * _L, _L)
                    xr_v[sl] = fi * _LROWS + lax.shift_right_logical(
                        xi_v[fi, pl.ds(loff + sb * _SB + j * _L, _L)], 7)
                    return cc

                lax.fori_loop(0, _SB // _L, _ridx, 0)
                pltpu.async_copy(lscr_hbm.at[xr_v], lr0_v if sem is se0
                                 else lr1_v, sem)

        def _ldrain(t, lr_v, xr_v, sem):
            @pl.when(t < _NLT)
            def _():
                fi = t // _NLSB
                sb = t - fi * _NLSB
                pltpu.make_async_copy(lscr_hbm.at[xr_v], lr_v, sem).wait()
                for j in range(_SB // _L):
                    sl16 = pl.ds(sb * _SB + j * _L, _L)
                    col = xi_v[fi, pl.ds(loff + sb * _SB + j * _L, _L)] \
                        & jnp.full((_L,), 127, jnp.int32)
                    rr = j * _L + lane
                    acc_v[0, sl16] = acc_v[0, sl16] + plsc.load_gather(
                        lr_v, [rr, col])

        _lfire(0, xr0_v, se0)
        _lfire(1, xr1_v, se1)

        def _lpair(i, carry):
            t0 = 2 * i
            _ldrain(t0, lr0_v, xr0_v, se0)
            _lfire(t0 + 2, xr0_v, se0)
            _ldrain(t0 + 1, lr1_v, xr1_v, se1)
            _lfire(t0 + 3, xr1_v, se1)
            return carry

        lax.fori_loop(0, _NLT // 2, _lpair, 0)
        pltpu.sync_copy(acc_v, out_hbm.at[pl.ds(N_FIELDS * DIM, 1),
                                          pl.ds(lbase, _LB)])

    pl.run_scoped(
        _phase_b,
        pltpu.VMEM((N_FIELDS, _BPT), jnp.int32),
        pltpu.VMEM((_SB,), jnp.int32),
        pltpu.VMEM((_SB,), jnp.int32),
        pltpu.VMEM((_SB, 8 * DIM), jnp.float32),
        pltpu.VMEM((_SB, 8 * DIM), jnp.float32),
        pltpu.VMEM((DIM, _BPT), jnp.float32),
        pltpu.VMEM((1, _LB), jnp.float32),
    )


@jax.jit
def _sc_call(xT, emb3, lin2):
    mesh = plsc.VectorSubcoreMesh(core_axis_name="c", subcore_axis_name="s")
    return pl.kernel(
        _sc_body,
        mesh=mesh,
        compiler_params=pltpu.CompilerParams(needs_layout_passes=False),
        out_type=jax.ShapeDtypeStruct((OUT_D, BATCH), jnp.float32),
        scratch_types=[
            pltpu.MemorySpace.HBM((N_FIELDS * VOCAB // 8, 8 * DIM),
                                  jnp.float32),
            pltpu.MemorySpace.HBM((N_FIELDS * _LROWS, 128), jnp.float32),
            pltpu.SemaphoreType.DMA,
            pltpu.SemaphoreType.DMA,
            pltpu.SemaphoreType.DMA,
            pltpu.SemaphoreType.DMA,
        ],
    )(xT, emb3, lin2)


def kernel(x, emb_tables, lin_tables):
    xT = x.T
    emb3 = jnp.transpose(emb_tables, (0, 2, 1))
    lin2 = lin_tables.reshape(N_FIELDS, VOCAB)
    outT = _sc_call(xT, emb3, lin2)
    return outT.T


# speedup vs baseline: 2.4252x; 1.0863x over previous
import jax
import jax.numpy as jnp
from jax import lax
from jax.experimental import pallas as pl
from jax.experimental.pallas import tpu as pltpu
from jax.experimental.pallas import tpu_sc as plsc

N_FIELDS = 26
VOCAB = 100000
DIM = 16
BATCH = 16384
OUT_D = N_FIELDS * DIM + 1

_info = plsc.get_sparse_core_info()
_NC, _NS, _L = _info.num_cores, _info.num_subcores, _info.num_lanes
_FPC = N_FIELDS // _NC
_BPT = BATCH // _NS
_NJ = _BPT // _L

_W = 512
_NFULL = VOCAB // _W
_WLAST = VOCAB - _NFULL * _W
_NWIN = _NFULL + 1
_NTASK = _FPC * _NWIN
_KMAX = (_NTASK + _NS - 1) // _NS

_SB = 128
_NSB = _BPT // _SB
_LB = BATCH // _NC // _NS
_NLSB = _LB // _SB
_LROWS = (VOCAB + 127) // 128 + 1
_W2 = 1024
_NW2 = (VOCAB + _W2 - 1) // _W2


def _sc_body(xT_hbm, emb_hbm, lin_hbm, out_hbm, scr_hbm, lscr_hbm, se0, se1,
             sw0, sw1):
    c = lax.axis_index("c")
    s = lax.axis_index("s")
    lane = lax.iota(jnp.int32, _L)
    lane_hi = lax.shift_right_logical(lane, 3)
    lane_lo16 = (lane & jnp.full((_L,), 7, jnp.int32)) * DIM

    def _phase_a(in0_v, in1_v, ob0_v, ob1_v, sa0, sa1, sb0, sb1):
        def _src(k):
            task = s + _NS * k
            f_l = task // _NWIN
            w = task - f_l * _NWIN
            f = c * _FPC + f_l
            return task, f, w, w * _W

        def _fire_in(k, in_v, sem):
            task, f, w, wlo = _src(k)

            @pl.when(task < _NTASK)
            def _():
                @pl.when(w < _NFULL)
                def _():
                    pltpu.async_copy(
                        emb_hbm.at[pl.ds(f, 1), :, pl.ds(wlo, _W)], in_v,
                        sem)

                @pl.when(w == _NFULL)
                def _():
                    pltpu.async_copy(
                        emb_hbm.at[pl.ds(f, 1), :, pl.ds(wlo, 256)],
                        in_v.at[:, :, pl.ds(0, 256)], sem)

        def _proc(k, in_v, ob_v, sem_in, sem_out):
            task, f, w, wlo = _src(k)

            @pl.when(task < _NTASK)
            def _():
                pl.when(w < _NFULL)(lambda: pltpu.make_async_copy(
                    emb_hbm.at[pl.ds(f, 1), :, pl.ds(wlo, _W)], in_v,
                    sem_in).wait())
                pl.when(w == _NFULL)(lambda: pltpu.make_async_copy(
                    emb_hbm.at[pl.ds(f, 1), :, pl.ds(wlo, 256)],
                    in_v.at[:, :, pl.ds(0, 256)], sem_in).wait())

            def _do(wr_ext):
                def _tr(j, cc):
                    v0 = j * _L
                    sl = pl.ds(v0, _L)
                    rr = v0 // 8 + lane_hi
                    for d in range(DIM):
                        plsc.store_scatter(ob_v, [rr, lane_lo16 + d],
                                           in_v[0, d, sl])
                    return cc

                lax.fori_loop(0, wr_ext // _L, _tr, 0)
                pltpu.async_copy(
                    ob_v.at[pl.ds(0, wr_ext // 8), :],
                    scr_hbm.at[pl.ds((f * VOCAB + wlo) // 8,
                                     wr_ext // 8), :], sem_out)

            def _wait_out(wr_ext):
                task2, f2, w2, wlo2 = _src(k - 2)
                pltpu.make_async_copy(
                    ob_v.at[pl.ds(0, wr_ext // 8), :],
                    scr_hbm.at[pl.ds((f2 * VOCAB + wlo2) // 8,
                                     wr_ext // 8), :], sem_out).wait()

            @pl.when(k >= 2)
            def _():
                _, _, w2, _ = _src(k - 2)
                pl.when(w2 < _NFULL)(lambda: _wait_out(_W))
                pl.when(w2 == _NFULL)(lambda: _wait_out(_WLAST))

            @pl.when(task < _NTASK)
            def _():
                pl.when(w < _NFULL)(lambda: _do(_W))
                pl.when(w == _NFULL)(lambda: _do(_WLAST))

        _fire_in(0, in0_v, sa0)
        _fire_in(1, in1_v, sa1)

        def _pairs(i, carry):
            k0 = 2 * i
            _proc(k0, in0_v, ob0_v, sa0, sb0)
            _fire_in(k0 + 2, in0_v, sa0)
            _proc(k0 + 1, in1_v, ob1_v, sa1, sb1)
            _fire_in(k0 + 3, in1_v, sa1)
            return carry

        lax.fori_loop(0, (_KMAX + 1) // 2, _pairs, 0)
        def _wait_last(k, ob_v, sem_out):
            task, f, w, wlo = _src(k)

            @pl.when(task < _NTASK)
            def _():
                def _w(wr_ext):
                    pltpu.make_async_copy(
                        ob_v.at[pl.ds(0, wr_ext // 8), :],
                        scr_hbm.at[pl.ds((f * VOCAB + wlo) // 8,
                                         wr_ext // 8), :], sem_out).wait()

                pl.when(w < _NFULL)(lambda: _w(_W))
                pl.when(w == _NFULL)(lambda: _w(_WLAST))

        klast = 2 * ((_KMAX + 1) // 2) - 1
        _wait_last(klast - 1, ob0_v, sb0)
        _wait_last(klast, ob1_v, sb1)

    pl.run_scoped(
        _phase_a,
        pltpu.VMEM((1, DIM, _W), jnp.float32),
        pltpu.VMEM((1, DIM, _W), jnp.float32),
        pltpu.VMEM((_W // 8, 8 * DIM), jnp.float32),
        pltpu.VMEM((_W // 8, 8 * DIM), jnp.float32),
        pltpu.SemaphoreType.DMA,
        pltpu.SemaphoreType.DMA,
        pltpu.SemaphoreType.DMA,
        pltpu.SemaphoreType.DMA,
    )

    def _phase_a2(li_v, ls_v):
        def _task(k, carry):
            task = s + _NS * k
            ntask = 4 * _NW2

            @pl.when(task < ntask)
            def _():
                g = task // _NW2
                w = task - g * _NW2
                wlo = w * _W2

                def _do(rd_ext, nw):
                    pltpu.sync_copy(
                        lin_hbm.at[pl.ds(8 * g, 8), pl.ds(wlo, rd_ext)],
                        li_v.at[:, pl.ds(0, rd_ext)])
                    nrow = jnp.where(g == 3, 2, 8)

                    def _row(r, cc):
                        f = 8 * g + r

                        @pl.when(r < nrow)
                        def _():
                            def _cp(kk, c2):
                                ls_v[kk // 8, pl.ds((kk % 8) * _L, _L)] = \
                                    li_v[r, pl.ds(kk * _L, _L)]
                                return c2

                            lax.fori_loop(0, nw * 8, _cp, 0)
                            pltpu.sync_copy(
                                ls_v.at[pl.ds(0, nw), :],
                                lscr_hbm.at[pl.ds(f * _LROWS + wlo // 128,
                                                  nw), :])

                        return cc

                    lax.fori_loop(0, 8, _row, 0)

                pl.when(w < _NW2 - 1)(lambda: _do(_W2, _W2 // 128))
                pl.when(w == _NW2 - 1)(lambda: _do(768, 6))

            return carry

        lax.fori_loop(0, (4 * _NW2 + _NS - 1) // _NS, _task, 0)

    pl.run_scoped(
        _phase_a2,
        pltpu.VMEM((8, _W2), jnp.float32),
        pltpu.VMEM((_W2 // 128, 128), jnp.float32),
    )
    plsc.subcore_barrier()

    def _phase_b(xi_v, xr0_v, xr1_v, lr0_v, lr1_v, tb0_v, acc_v):
        base = s * _BPT
        pltpu.sync_copy(xT_hbm.at[:, pl.ds(base, _BPT)], xi_v)
        loff = c * _LB
        lbase = base + loff

        def _field_body(f_l, carry):
            f = c * _FPC + f_l
            tb_v, sem_w = tb0_v, sw0

            @pl.when(f_l >= 1)
            def _():
                pltpu.make_async_copy(
                    tb_v, out_hbm.at[pl.ds((f - 1) * DIM, DIM),
                                     pl.ds(base, _BPT)], sem_w).wait()

            def _gfire(sb, lr_v, xr_v, sem):
                def _ridx(j, cc):
                    sl = pl.ds(j * _L, _L)
                    xr_v[sl] = lax.shift_right_logical(
                        xi_v[f, pl.ds(sb * _SB + j * _L, _L)] + f * VOCAB,
                        3)
                    return cc

                lax.fori_loop(0, _SB // _L, _ridx, 0)
                pltpu.async_copy(scr_hbm.at[xr_v], lr_v, sem)

            def _gdrain(sb, lr_v, xr_v, sem):
                pltpu.make_async_copy(scr_hbm.at[xr_v], lr_v, sem).wait()
                for j in range(_SB // _L):
                    sl16 = pl.ds(sb * _SB + j * _L, _L)
                    vm = (xi_v[f, sl16] & jnp.full((_L,), 7,
                                                   jnp.int32)) * DIM
                    rr = j * _L + lane
                    for d in range(DIM):
                        tb_v[d, sl16] = plsc.load_gather(lr_v, [rr, vm + d])

            _gfire(0, lr0_v, xr0_v, se0)
            _gfire(1, lr1_v, xr1_v, se1)
            for sb in range(_NSB):
                _gdrain(sb, lr0_v if sb % 2 == 0 else lr1_v,
                        xr0_v if sb % 2 == 0 else xr1_v,
                        se0 if sb % 2 == 0 else se1)
                if sb + 2 < _NSB:
                    _gfire(sb + 2, lr0_v if sb % 2 == 0 else lr1_v,
                           xr0_v if sb % 2 == 0 else xr1_v,
                           se0 if sb % 2 == 0 else se1)

            pltpu.async_copy(tb_v, out_hbm.at[pl.ds(f * DIM, DIM),
                                              pl.ds(base, _BPT)], sem_w)
            return carry

        lax.fori_loop(0, _FPC, _field_body, 0)
        f_last = c * _FPC + _FPC - 1
        pltpu.make_async_copy(
            tb0_v, out_hbm.at[pl.ds(f_last * DIM, DIM),
                              pl.ds(base, _BPT)], sw0).wait()

        def _zero(j, carry):
            acc_v[0, pl.ds(j * _L, _L)] = jnp.zeros((_L,), jnp.float32)
            return carry

        lax.fori_loop(0, _LB // _L, _zero, 0)

        _NLT = N_FIELDS * _NLSB

        def _lfire(t, xr_v, sem):
            @pl.when(t < _NLT)
            def _():
                fi = t // _NLSB
                sb = t - fi * _NLSB

                def _ridx(j, cc):
                    sl = pl.ds(j * _L, _L)
                    xr_v[sl] = fi * _LROWS + lax.shift_right_logical(
                        xi_v[fi, pl.ds(loff + sb * _SB + j * _L, _L)], 7)
                    return cc

                lax.fori_loop(0, _SB // _L, _ridx, 0)
                pltpu.async_copy(lscr_hbm.at[xr_v], lr0_v if sem is se0
                                 else lr1_v, sem)

        def _ldrain(t, lr_v, xr_v, sem):
            @pl.when(t < _NLT)
            def _():
                fi = t // _NLSB
                sb = t - fi * _NLSB
                pltpu.make_async_copy(lscr_hbm.at[xr_v], lr_v, sem).wait()
                for j in range(_SB // _L):
                    sl16 = pl.ds(sb * _SB + j * _L, _L)
                    col = xi_v[fi, pl.ds(loff + sb * _SB + j * _L, _L)] \
                        & jnp.full((_L,), 127, jnp.int32)
                    rr = j * _L + lane
                    acc_v[0, sl16] = acc_v[0, sl16] + plsc.load_gather(
                        lr_v, [rr, col])

        _lfire(0, xr0_v, se0)
        _lfire(1, xr1_v, se1)

        def _lpair(i, carry):
            t0 = 2 * i
            _ldrain(t0, lr0_v, xr0_v, se0)
            _lfire(t0 + 2, xr0_v, se0)
            _ldrain(t0 + 1, lr1_v, xr1_v, se1)
            _lfire(t0 + 3, xr1_v, se1)
            return carry

        lax.fori_loop(0, _NLT // 2, _lpair, 0)
        pltpu.sync_copy(acc_v, out_hbm.at[pl.ds(N_FIELDS * DIM, 1),
                                          pl.ds(lbase, _LB)])

    pl.run_scoped(
        _phase_b,
        pltpu.VMEM((N_FIELDS, _BPT), jnp.int32),
        pltpu.VMEM((_SB,), jnp.int32),
        pltpu.VMEM((_SB,), jnp.int32),
        pltpu.VMEM((_SB, 8 * DIM), jnp.float32),
        pltpu.VMEM((_SB, 8 * DIM), jnp.float32),
        pltpu.VMEM((DIM, _BPT), jnp.float32),
        pltpu.VMEM((1, _LB), jnp.float32),
    )


@jax.jit
def _sc_call(xT, emb3, lin2):
    mesh = plsc.VectorSubcoreMesh(core_axis_name="c", subcore_axis_name="s")
    return pl.kernel(
        _sc_body,
        mesh=mesh,
        compiler_params=pltpu.CompilerParams(needs_layout_passes=False),
        out_type=jax.ShapeDtypeStruct((OUT_D, BATCH), jnp.float32),
        scratch_types=[
            pltpu.MemorySpace.HBM((N_FIELDS * VOCAB // 8, 8 * DIM),
                                  jnp.float32),
            pltpu.MemorySpace.HBM((N_FIELDS * _LROWS, 128), jnp.float32),
            pltpu.SemaphoreType.DMA,
            pltpu.SemaphoreType.DMA,
            pltpu.SemaphoreType.DMA,
            pltpu.SemaphoreType.DMA,
        ],
    )(xT, emb3, lin2)


def kernel(x, emb_tables, lin_tables):
    xT = x.T
    emb3 = jnp.transpose(emb_tables, (0, 2, 1))
    lin2 = lin_tables.reshape(N_FIELDS, VOCAB)
    outT = _sc_call(xT, emb3, lin2)
    return outT.T


# speedup vs baseline: 2.4290x; 1.0015x over previous
import jax
import jax.numpy as jnp
from jax import lax
from jax.experimental import pallas as pl
from jax.experimental.pallas import tpu as pltpu
from jax.experimental.pallas import tpu_sc as plsc

N_FIELDS = 26
VOCAB = 100000
DIM = 16
BATCH = 16384
OUT_D = N_FIELDS * DIM + 1

_info = plsc.get_sparse_core_info()
_NC, _NS, _L = _info.num_cores, _info.num_subcores, _info.num_lanes
_FPC = N_FIELDS // _NC
_BPT = BATCH // _NS
_NJ = _BPT // _L

_W = 512
_NFULL = VOCAB // _W
_WLAST = VOCAB - _NFULL * _W
_NWIN = _NFULL + 1
_NTASK = _FPC * _NWIN
_KMAX = (_NTASK + _NS - 1) // _NS

_SB = 128
_NSB = _BPT // _SB
_LB = BATCH // _NC // _NS
_NLSB = _LB // _SB
_LROWS = (VOCAB + 127) // 128 + 1
_W2 = 1024
_NW2 = (VOCAB + _W2 - 1) // _W2


def _sc_body(xT_hbm, emb_hbm, lin_hbm, out_hbm, scr_hbm, lscr_hbm, se0, se1,
             sw0, sw1):
    c = lax.axis_index("c")
    s = lax.axis_index("s")
    lane = lax.iota(jnp.int32, _L)
    lane_hi = lax.shift_right_logical(lane, 3)
    lane_lo16 = (lane & jnp.full((_L,), 7, jnp.int32)) * DIM

    def _phase_a(in0_v, in1_v, ob0_v, ob1_v, sa0, sa1, sb0, sb1):
        def _src(k):
            task = s + _NS * k
            f_l = task // _NWIN
            w = task - f_l * _NWIN
            f = c * _FPC + f_l
            return task, f, w, w * _W

        def _fire_in(k, in_v, sem):
            task, f, w, wlo = _src(k)

            @pl.when(task < _NTASK)
            def _():
                @pl.when(w < _NFULL)
                def _():
                    pltpu.async_copy(
                        emb_hbm.at[pl.ds(f, 1), :, pl.ds(wlo, _W)], in_v,
                        sem)

                @pl.when(w == _NFULL)
                def _():
                    pltpu.async_copy(
                        emb_hbm.at[pl.ds(f, 1), :, pl.ds(wlo, 256)],
                        in_v.at[:, :, pl.ds(0, 256)], sem)

        def _proc(k, in_v, ob_v, sem_in, sem_out):
            task, f, w, wlo = _src(k)

            @pl.when(task < _NTASK)
            def _():
                pl.when(w < _NFULL)(lambda: pltpu.make_async_copy(
                    emb_hbm.at[pl.ds(f, 1), :, pl.ds(wlo, _W)], in_v,
                    sem_in).wait())
                pl.when(w == _NFULL)(lambda: pltpu.make_async_copy(
                    emb_hbm.at[pl.ds(f, 1), :, pl.ds(wlo, 256)],
                    in_v.at[:, :, pl.ds(0, 256)], sem_in).wait())

            def _do(wr_ext):
                def _tr(j, cc):
                    for u in range(2):
                        v0 = (2 * j + u) * _L
                        sl = pl.ds(v0, _L)
                        rr = v0 // 8 + lane_hi
                        for d in range(DIM):
                            plsc.store_scatter(ob_v, [rr, lane_lo16 + d],
                                               in_v[0, d, sl])
                    return cc

                lax.fori_loop(0, wr_ext // (2 * _L), _tr, 0)
                pltpu.async_copy(
                    ob_v.at[pl.ds(0, wr_ext // 8), :],
                    scr_hbm.at[pl.ds((f * VOCAB + wlo) // 8,
                                     wr_ext // 8), :], sem_out)

            def _wait_out(wr_ext):
                task2, f2, w2, wlo2 = _src(k - 2)
                pltpu.make_async_copy(
                    ob_v.at[pl.ds(0, wr_ext // 8), :],
                    scr_hbm.at[pl.ds((f2 * VOCAB + wlo2) // 8,
                                     wr_ext // 8), :], sem_out).wait()

            @pl.when(k >= 2)
            def _():
                _, _, w2, _ = _src(k - 2)
                pl.when(w2 < _NFULL)(lambda: _wait_out(_W))
                pl.when(w2 == _NFULL)(lambda: _wait_out(_WLAST))

            @pl.when(task < _NTASK)
            def _():
                pl.when(w < _NFULL)(lambda: _do(_W))
                pl.when(w == _NFULL)(lambda: _do(_WLAST))

        _fire_in(0, in0_v, sa0)
        _fire_in(1, in1_v, sa1)

        def _pairs(i, carry):
            k0 = 2 * i
            _proc(k0, in0_v, ob0_v, sa0, sb0)
            _fire_in(k0 + 2, in0_v, sa0)
            _proc(k0 + 1, in1_v, ob1_v, sa1, sb1)
            _fire_in(k0 + 3, in1_v, sa1)
            return carry

        lax.fori_loop(0, (_KMAX + 1) // 2, _pairs, 0)
        def _wait_last(k, ob_v, sem_out):
            task, f, w, wlo = _src(k)

            @pl.when(task < _NTASK)
            def _():
                def _w(wr_ext):
                    pltpu.make_async_copy(
                        ob_v.at[pl.ds(0, wr_ext // 8), :],
                        scr_hbm.at[pl.ds((f * VOCAB + wlo) // 8,
                                         wr_ext // 8), :], sem_out).wait()

                pl.when(w < _NFULL)(lambda: _w(_W))
                pl.when(w == _NFULL)(lambda: _w(_WLAST))

        klast = 2 * ((_KMAX + 1) // 2) - 1
        _wait_last(klast - 1, ob0_v, sb0)
        _wait_last(klast, ob1_v, sb1)

    pl.run_scoped(
        _phase_a,
        pltpu.VMEM((1, DIM, _W), jnp.float32),
        pltpu.VMEM((1, DIM, _W), jnp.float32),
        pltpu.VMEM((_W // 8, 8 * DIM), jnp.float32),
        pltpu.VMEM((_W // 8, 8 * DIM), jnp.float32),
        pltpu.SemaphoreType.DMA,
        pltpu.SemaphoreType.DMA,
        pltpu.SemaphoreType.DMA,
        pltpu.SemaphoreType.DMA,
    )

    def _phase_a2(li_v, ls_v):
        def _task(k, carry):
            task = s + _NS * k
            ntask = 4 * _NW2

            @pl.when(task < ntask)
            def _():
                g = task // _NW2
                w = task - g * _NW2
                wlo = w * _W2

                def _do(rd_ext, nw):
                    pltpu.sync_copy(
                        lin_hbm.at[pl.ds(8 * g, 8), pl.ds(wlo, rd_ext)],
                        li_v.at[:, pl.ds(0, rd_ext)])
                    nrow = jnp.where(g == 3, 2, 8)

                    def _row(r, cc):
                        f = 8 * g + r

                        @pl.when(r < nrow)
                        def _():
                            def _cp(kk, c2):
                                ls_v[kk // 8, pl.ds((kk % 8) * _L, _L)] = \
                                    li_v[r, pl.ds(kk * _L, _L)]
                                return c2

                            lax.fori_loop(0, nw * 8, _cp, 0)
                            pltpu.sync_copy(
                                ls_v.at[pl.ds(0, nw), :],
                                lscr_hbm.at[pl.ds(f * _LROWS + wlo // 128,
                                                  nw), :])

                        return cc

                    lax.fori_loop(0, 8, _row, 0)

                pl.when(w < _NW2 - 1)(lambda: _do(_W2, _W2 // 128))
                pl.when(w == _NW2 - 1)(lambda: _do(768, 6))

            return carry

        lax.fori_loop(0, (4 * _NW2 + _NS - 1) // _NS, _task, 0)

    pl.run_scoped(
        _phase_a2,
        pltpu.VMEM((8, _W2), jnp.float32),
        pltpu.VMEM((_W2 // 128, 128), jnp.float32),
    )
    plsc.subcore_barrier()

    def _phase_b(xi_v, xr0_v, xr1_v, lr0_v, lr1_v, tb0_v, acc_v):
        base = s * _BPT
        pltpu.sync_copy(xT_hbm.at[:, pl.ds(base, _BPT)], xi_v)
        loff = c * _LB
        lbase = base + loff

        def _field_body(f_l, carry):
            f = c * _FPC + f_l
            tb_v, sem_w = tb0_v, sw0

            @pl.when(f_l >= 1)
            def _():
                pltpu.make_async_copy(
                    tb_v, out_hbm.at[pl.ds((f - 1) * DIM, DIM),
                                     pl.ds(base, _BPT)], sem_w).wait()

            def _gfire(sb, lr_v, xr_v, sem):
                def _ridx(j, cc):
                    sl = pl.ds(j * _L, _L)
                    xr_v[sl] = lax.shift_right_logical(
                        xi_v[f, pl.ds(sb * _SB + j * _L, _L)] + f * VOCAB,
                        3)
                    return cc

                lax.fori_loop(0, _SB // _L, _ridx, 0)
                pltpu.async_copy(scr_hbm.at[xr_v], lr_v, sem)

            def _gdrain(sb, lr_v, xr_v, sem):
                pltpu.make_async_copy(scr_hbm.at[xr_v], lr_v, sem).wait()
                for j in range(_SB // _L):
                    sl16 = pl.ds(sb * _SB + j * _L, _L)
                    vm = (xi_v[f, sl16] & jnp.full((_L,), 7,
                                                   jnp.int32)) * DIM
                    rr = j * _L + lane
                    for d in range(DIM):
                        tb_v[d, sl16] = plsc.load_gather(lr_v, [rr, vm + d])

            _gfire(0, lr0_v, xr0_v, se0)
            _gfire(1, lr1_v, xr1_v, se1)
            for sb in range(_NSB):
                _gdrain(sb, lr0_v if sb % 2 == 0 else lr1_v,
                        xr0_v if sb % 2 == 0 else xr1_v,
                        se0 if sb % 2 == 0 else se1)
                if sb + 2 < _NSB:
                    _gfire(sb + 2, lr0_v if sb % 2 == 0 else lr1_v,
                           xr0_v if sb % 2 == 0 else xr1_v,
                           se0 if sb % 2 == 0 else se1)

            pltpu.async_copy(tb_v, out_hbm.at[pl.ds(f * DIM, DIM),
                                              pl.ds(base, _BPT)], sem_w)
            return carry

        lax.fori_loop(0, _FPC, _field_body, 0)
        f_last = c * _FPC + _FPC - 1
        pltpu.make_async_copy(
            tb0_v, out_hbm.at[pl.ds(f_last * DIM, DIM),
                              pl.ds(base, _BPT)], sw0).wait()

        def _zero(j, carry):
            acc_v[0, pl.ds(j * _L, _L)] = jnp.zeros((_L,), jnp.float32)
            return carry

        lax.fori_loop(0, _LB // _L, _zero, 0)

        _NLT = N_FIELDS * _NLSB

        def _lfire(t, xr_v, sem):
            @pl.when(t < _NLT)
            def _():
                fi = t // _NLSB
                sb = t - fi * _NLSB

                def _ridx(j, cc):
                    sl = pl.ds(j * _L, _L)
                    xr_v[sl] = fi * _LROWS + lax.shift_right_logical(
                        xi_v[fi, pl.ds(loff + sb * _SB + j * _L, _L)], 7)
                    return cc

                lax.fori_loop(0, _SB // _L, _ridx, 0)
                pltpu.async_copy(lscr_hbm.at[xr_v], lr0_v if sem is se0
                                 else lr1_v, sem)

        def _ldrain(t, lr_v, xr_v, sem):
            @pl.when(t < _NLT)
            def _():
                fi = t // _NLSB
                sb = t - fi * _NLSB
                pltpu.make_async_copy(lscr_hbm.at[xr_v], lr_v, sem).wait()
                for j in range(_SB // _L):
                    sl16 = pl.ds(sb * _SB + j * _L, _L)
                    col = xi_v[fi, pl.ds(loff + sb * _SB + j * _L, _L)] \
                        & jnp.full((_L,), 127, jnp.int32)
                    rr = j * _L + lane
                    acc_v[0, sl16] = acc_v[0, sl16] + plsc.load_gather(
                        lr_v, [rr, col])

        _lfire(0, xr0_v, se0)
        _lfire(1, xr1_v, se1)

        def _lpair(i, carry):
            t0 = 2 * i
            _ldrain(t0, lr0_v, xr0_v, se0)
            _lfire(t0 + 2, xr0_v, se0)
            _ldrain(t0 + 1, lr1_v, xr1_v, se1)
            _lfire(t0 + 3, xr1_v, se1)
            return carry

        lax.fori_loop(0, _NLT // 2, _lpair, 0)
        pltpu.sync_copy(acc_v, out_hbm.at[pl.ds(N_FIELDS * DIM, 1),
                                          pl.ds(lbase, _LB)])

    pl.run_scoped(
        _phase_b,
        pltpu.VMEM((N_FIELDS, _BPT), jnp.int32),
        pltpu.VMEM((_SB,), jnp.int32),
        pltpu.VMEM((_SB,), jnp.int32),
        pltpu.VMEM((_SB, 8 * DIM), jnp.float32),
        pltpu.VMEM((_SB, 8 * DIM), jnp.float32),
        pltpu.VMEM((DIM, _BPT), jnp.float32),
        pltpu.VMEM((1, _LB), jnp.float32),
    )


@jax.jit
def _sc_call(xT, emb3, lin2):
    mesh = plsc.VectorSubcoreMesh(core_axis_name="c", subcore_axis_name="s")
    return pl.kernel(
        _sc_body,
        mesh=mesh,
        compiler_params=pltpu.CompilerParams(needs_layout_passes=False),
        out_type=jax.ShapeDtypeStruct((OUT_D, BATCH), jnp.float32),
        scratch_types=[
            pltpu.MemorySpace.HBM((N_FIELDS * VOCAB // 8, 8 * DIM),
                                  jnp.float32),
            pltpu.MemorySpace.HBM((N_FIELDS * _LROWS, 128), jnp.float32),
            pltpu.SemaphoreType.DMA,
            pltpu.SemaphoreType.DMA,
            pltpu.SemaphoreType.DMA,
            pltpu.SemaphoreType.DMA,
        ],
    )(xT, emb3, lin2)


def kernel(x, emb_tables, lin_tables):
    xT = x.T
    emb3 = jnp.transpose(emb_tables, (0, 2, 1))
    lin2 = lin_tables.reshape(N_FIELDS, VOCAB)
    outT = _sc_call(xT, emb3, lin2)
    return outT.T
